# Initial kernel scaffold; baseline (speedup 1.0000x reference)
import jax
import jax.numpy as jnp
from jax.experimental import pallas as pl


def kernel(x, edge_index, edge_attr, W1_0, b1_0, gamma_0, beta_0, W2_0, b2_0, eps_0, W1_1, b1_1, gamma_1, beta_1, W2_1, b2_1, eps_1, W1_2, b1_2, gamma_2, beta_2, W2_2, b2_2, eps_2, W_lin, b_lin):
    raise NotImplementedError("write your pallas kernel here")


# speedup vs baseline: 2.9947x; 2.9947x over previous
import functools

import jax
import jax.numpy as jnp
from jax import lax
from jax.experimental import pallas as pl
from jax.experimental.pallas import tpu as pltpu
from jax.experimental.pallas import tpu_sc as plsc

N = 10000
E = 320000
D = 128
NC = 2
NS = 16
NW = NC * NS
EPT = E // NW
C = 80
NCHUNK = EPT // C
ZCH = 80
NZ = N // ZCH
VPR = D // 16


def _edge_body(h_hbm, src_hbm, dst_hbm, attr_hbm, out_hbm,
               sidx, didx, gbuf, abuf, agg_sh, sem):
    c = lax.axis_index("c")
    s = lax.axis_index("s")
    wid = s * NC + c
    base = wid * EPT
    nz_mine = (NZ - s + NS - 1) // NS

    def _zrow(i, carry):
        for j in range(VPR):
            gbuf[i, pl.ds(j * 16, 16)] = jnp.zeros((16,), jnp.float32)
        return carry
    lax.fori_loop(0, ZCH, _zrow, 0)

    def _zcopy(k, carry):
        r = (s + k * NS) * ZCH
        pltpu.sync_copy(gbuf, agg_sh.at[pl.ds(r, ZCH)])
        return carry
    lax.fori_loop(0, nz_mine, _zcopy, 0)
    plsc.subcore_barrier()

    def _chunk(k, carry):
        off = base + k * C
        pltpu.sync_copy(src_hbm.at[pl.ds(off, C)], sidx)
        pltpu.sync_copy(attr_hbm.at[pl.ds(off, C)], abuf)
        pltpu.async_copy(h_hbm.at[sidx], gbuf, sem).wait()

        def _row(i, rc):
            for j in range(VPR):
                v = gbuf[i, pl.ds(j * 16, 16)] + abuf[i, pl.ds(j * 16, 16)]
                gbuf[i, pl.ds(j * 16, 16)] = jnp.maximum(v, 0.0)
            return rc
        lax.fori_loop(0, C, _row, 0)

        pltpu.sync_copy(dst_hbm.at[pl.ds(off, C)], didx)
        pltpu.sync_copy(gbuf, agg_sh.at[didx], add=True)
        return carry
    lax.fori_loop(0, NCHUNK, _chunk, 0)
    plsc.subcore_barrier()

    def _ocopy(k, carry):
        r = (s + k * NS) * ZCH
        pltpu.sync_copy(agg_sh.at[pl.ds(r, ZCH)],
                        out_hbm.at[pl.ds(c * N + r, ZCH)])
        return carry
    lax.fori_loop(0, nz_mine, _ocopy, 0)


@functools.lru_cache(maxsize=None)
def _get_edge_agg():
    return pl.kernel(
        _edge_body,
        out_type=jax.ShapeDtypeStruct((2 * N, D), jnp.float32),
        mesh=plsc.VectorSubcoreMesh(core_axis_name="c", subcore_axis_name="s"),
        scratch_types=[
            pltpu.VMEM((C,), jnp.int32),
            pltpu.VMEM((C,), jnp.int32),
            pltpu.VMEM((C, D), jnp.float32),
            pltpu.VMEM((C, D), jnp.float32),
            pltpu.VMEM_SHARED((N, D), jnp.float32),
            pltpu.SemaphoreType.DMA,
        ],
    )


def _dense_body(x_ref, agg_ref, eps_ref, w1_ref, b1_ref, g_ref, be_ref,
                w2_ref, b2_ref, o_ref):
    h = (1.0 + eps_ref[0, 0]) * x_ref[...] + agg_ref[0:N] + agg_ref[N:2 * N]
    h1 = jnp.dot(h, w1_ref[...], preferred_element_type=jnp.float32) + b1_ref[...]
    mu = jnp.mean(h1, axis=0, keepdims=True)
    var = jnp.mean(jnp.square(h1 - mu), axis=0, keepdims=True)
    hn = (h1 - mu) * (g_ref[...] * lax.rsqrt(var + 1e-5)) + be_ref[...]
    hr = jnp.maximum(hn, 0.0)
    h2 = jnp.dot(hr, w2_ref[...], preferred_element_type=jnp.float32) + b2_ref[...]
    o_ref[...] = jnp.maximum(h2, 0.0)


_dense = pl.pallas_call(
    _dense_body,
    out_shape=jax.ShapeDtypeStruct((N, D), jnp.float32),
)


def _final_body(h0_ref, h1_ref, h2_ref, h3_ref, wl_ref, bl_ref, o_ref):
    acc = jnp.dot(h0_ref[...], wl_ref[0:D], preferred_element_type=jnp.float32)
    acc += jnp.dot(h1_ref[...], wl_ref[D:2 * D], preferred_element_type=jnp.float32)
    acc += jnp.dot(h2_ref[...], wl_ref[2 * D:3 * D], preferred_element_type=jnp.float32)
    acc += jnp.dot(h3_ref[...], wl_ref[3 * D:4 * D], preferred_element_type=jnp.float32)
    o_ref[...] = acc + bl_ref[...]


_final = pl.pallas_call(
    _final_body,
    out_shape=jax.ShapeDtypeStruct((N, D), jnp.float32),
)


def kernel(x, edge_index, edge_attr,
           W1_0, b1_0, gamma_0, beta_0, W2_0, b2_0, eps_0,
           W1_1, b1_1, gamma_1, beta_1, W2_1, b2_1, eps_1,
           W1_2, b1_2, gamma_2, beta_2, W2_2, b2_2, eps_2,
           W_lin, b_lin):
    src = edge_index[0]
    dst = edge_index[1]
    params = [
        (W1_0, b1_0, gamma_0, beta_0, W2_0, b2_0, eps_0),
        (W1_1, b1_1, gamma_1, beta_1, W2_1, b2_1, eps_1),
        (W1_2, b1_2, gamma_2, beta_2, W2_2, b2_2, eps_2),
    ]
    h = x
    hs = [x]
    for l in range(3):
        W1, b1, gamma, beta, W2, b2, eps = params[l]
        agg = _get_edge_agg()(h, src, dst, edge_attr)
        h = _dense(h, agg, eps.reshape(1, 1), W1, b1.reshape(1, 2 * D),
                   gamma.reshape(1, 2 * D), beta.reshape(1, 2 * D),
                   W2, b2.reshape(1, D))
        hs.append(h)
    return _final(hs[0], hs[1], hs[2], hs[3], W_lin, b_lin.reshape(1, D))


# speedup vs baseline: 6.2090x; 2.0733x over previous
import functools

import jax
import jax.numpy as jnp
from jax import lax
from jax.experimental import pallas as pl
from jax.experimental.pallas import tpu as pltpu
from jax.experimental.pallas import tpu_sc as plsc

N = 10000
E = 320000
D = 128
NC = 2
NS = 16
NW = NC * NS
EPT = E // NW
C = 80
NCHUNK = EPT // C
ZCH = 80
NZ = N // ZCH
VPR = D // 16


def _edge_body(h_hbm, src_hbm, dst_hbm, attr_hbm, out_hbm,
               sidx, di0, di1, gb0, gb1, abuf, agg_sh,
               gs0, gs1, asem, ds0, ds1):
    c = lax.axis_index("c")
    s = lax.axis_index("s")
    wid = s * NC + c
    base = wid * EPT
    nz_mine = (NZ - s + NS - 1) // NS

    gbufs = (gb0, gb1)
    didxs = (di0, di1)
    gsems = (gs0, gs1)
    dsems = (ds0, ds1)

    pltpu.sync_copy(src_hbm.at[pl.ds(base, EPT)], sidx)

    def _zrow(i, carry):
        for j in range(VPR):
            gb0[i, pl.ds(j * 16, 16)] = jnp.zeros((16,), jnp.float32)
        return carry
    lax.fori_loop(0, ZCH, _zrow, 0)

    def _zcopy(k, carry):
        r = (s + k * NS) * ZCH
        pltpu.sync_copy(gb0, agg_sh.at[pl.ds(r, ZCH)])
        return carry
    lax.fori_loop(0, nz_mine, _zcopy, 0)
    plsc.subcore_barrier()

    def _issue_gd(q, b):
        pltpu.async_copy(dst_hbm.at[pl.ds(base + q * C, C)], didxs[b], dsems[b])
        pltpu.async_copy(h_hbm.at[sidx.at[pl.ds(q * C, C)]], gbufs[b], gsems[b])

    def _issue_attr(q):
        pltpu.async_copy(attr_hbm.at[pl.ds(base + q * C, C)], abuf, asem)

    def _step(q, b, prefetch):
        if prefetch:
            _issue_gd(q + 1, b ^ 1)
        pltpu.make_async_copy(attr_hbm.at[pl.ds(base + q * C, C)],
                              abuf, asem).wait()
        pltpu.make_async_copy(h_hbm.at[sidx.at[pl.ds(q * C, C)]],
                              gbufs[b], gsems[b]).wait()
        gbuf = gbufs[b]

        def _row(i, rc):
            for j in range(VPR):
                v = gbuf[i, pl.ds(j * 16, 16)] + abuf[i, pl.ds(j * 16, 16)]
                gbuf[i, pl.ds(j * 16, 16)] = jnp.maximum(v, 0.0)
            return rc
        lax.fori_loop(0, C, _row, 0)
        if prefetch:
            _issue_attr(q + 1)
        pltpu.make_async_copy(dst_hbm.at[pl.ds(base + q * C, C)],
                              didxs[b], dsems[b]).wait()
        pltpu.sync_copy(gbuf, agg_sh.at[didxs[b]], add=True)

    _issue_gd(0, 0)
    _issue_attr(0)

    def _super(j, carry):
        _step(2 * j, 0, True)
        _step(2 * j + 1, 1, True)
        return carry
    lax.fori_loop(0, (NCHUNK - 1) // 2, _super, 0)
    _step(NCHUNK - 1, 0, False)
    plsc.subcore_barrier()

    def _ocopy(k, carry):
        r = (s + k * NS) * ZCH
        pltpu.sync_copy(agg_sh.at[pl.ds(r, ZCH)],
                        out_hbm.at[pl.ds(c * N + r, ZCH)])
        return carry
    lax.fori_loop(0, nz_mine, _ocopy, 0)


@functools.lru_cache(maxsize=None)
def _get_edge_agg():
    return pl.kernel(
        _edge_body,
        out_type=jax.ShapeDtypeStruct((2 * N, D), jnp.float32),
        mesh=plsc.VectorSubcoreMesh(core_axis_name="c", subcore_axis_name="s"),
        scratch_types=[
            pltpu.VMEM((EPT,), jnp.int32),
            pltpu.VMEM((C,), jnp.int32),
            pltpu.VMEM((C,), jnp.int32),
            pltpu.VMEM((C, D), jnp.float32),
            pltpu.VMEM((C, D), jnp.float32),
            pltpu.VMEM((C, D), jnp.float32),
            pltpu.VMEM_SHARED((N, D), jnp.float32),
            pltpu.SemaphoreType.DMA,
            pltpu.SemaphoreType.DMA,
            pltpu.SemaphoreType.DMA,
            pltpu.SemaphoreType.DMA,
            pltpu.SemaphoreType.DMA,
        ],
    )


def _dense_body(x_ref, agg_ref, eps_ref, w1_ref, b1_ref, g_ref, be_ref,
                w2_ref, b2_ref, o_ref):
    h = (1.0 + eps_ref[0, 0]) * x_ref[...] + agg_ref[0:N] + agg_ref[N:2 * N]
    h1 = jnp.dot(h, w1_ref[...], preferred_element_type=jnp.float32) + b1_ref[...]
    mu = jnp.mean(h1, axis=0, keepdims=True)
    var = jnp.mean(jnp.square(h1 - mu), axis=0, keepdims=True)
    hn = (h1 - mu) * (g_ref[...] * lax.rsqrt(var + 1e-5)) + be_ref[...]
    hr = jnp.maximum(hn, 0.0)
    h2 = jnp.dot(hr, w2_ref[...], preferred_element_type=jnp.float32) + b2_ref[...]
    o_ref[...] = jnp.maximum(h2, 0.0)


_dense = pl.pallas_call(
    _dense_body,
    out_shape=jax.ShapeDtypeStruct((N, D), jnp.float32),
)


def _final_body(h0_ref, h1_ref, h2_ref, h3_ref, wl_ref, bl_ref, o_ref):
    acc = jnp.dot(h0_ref[...], wl_ref[0:D], preferred_element_type=jnp.float32)
    acc += jnp.dot(h1_ref[...], wl_ref[D:2 * D], preferred_element_type=jnp.float32)
    acc += jnp.dot(h2_ref[...], wl_ref[2 * D:3 * D], preferred_element_type=jnp.float32)
    acc += jnp.dot(h3_ref[...], wl_ref[3 * D:4 * D], preferred_element_type=jnp.float32)
    o_ref[...] = acc + bl_ref[...]


_final = pl.pallas_call(
    _final_body,
    out_shape=jax.ShapeDtypeStruct((N, D), jnp.float32),
)


def kernel(x, edge_index, edge_attr,
           W1_0, b1_0, gamma_0, beta_0, W2_0, b2_0, eps_0,
           W1_1, b1_1, gamma_1, beta_1, W2_1, b2_1, eps_1,
           W1_2, b1_2, gamma_2, beta_2, W2_2, b2_2, eps_2,
           W_lin, b_lin):
    src = edge_index[0]
    dst = edge_index[1]
    params = [
        (W1_0, b1_0, gamma_0, beta_0, W2_0, b2_0, eps_0),
        (W1_1, b1_1, gamma_1, beta_1, W2_1, b2_1, eps_1),
        (W1_2, b1_2, gamma_2, beta_2, W2_2, b2_2, eps_2),
    ]
    h = x
    hs = [x]
    for l in range(3):
        W1, b1, gamma, beta, W2, b2, eps = params[l]
        agg = _get_edge_agg()(h, src, dst, edge_attr)
        h = _dense(h, agg, eps.reshape(1, 1), W1, b1.reshape(1, 2 * D),
                   gamma.reshape(1, 2 * D), beta.reshape(1, 2 * D),
                   W2, b2.reshape(1, D))
        hs.append(h)
    return _final(hs[0], hs[1], hs[2], hs[3], W_lin, b_lin.reshape(1, D))
